# baseline (speedup 1.0000x reference)
import functools

import jax
import jax.numpy as jnp
from jax import lax
from jax.experimental import pallas as pl
from jax.experimental.pallas import tpu as pltpu
from jax.experimental.pallas import tpu_sc as plsc

N_NODES = 100000
NUM_GRAPHS = 64
NW = 32
CHUNK = 3136
LAST = NW - 1
TAIL = N_NODES - LAST * CHUNK
TABLE_PAD = 128
L = 16
VECS = CHUNK // L
TAIL_VECS = TAIL // L
FCHUNK = CHUNK * 3
FTAIL = TAIL * 3


def _sc_body(ne_hbm, sp_hbm, lg_hbm, bt_hbm, tab_hbm, f_hbm,
             oe_hbm, par_hbm, fo_hbm,
             ne_v, sp_v, lg_v, bt_v, oe_v, tab_v, acc_v, par_v, f_v,
             sem, fsem):
    wid = lax.axis_index("s") * 2 + lax.axis_index("c")
    base = wid * CHUNK
    is_last = wid == LAST

    tab_cp = pltpu.async_copy(tab_hbm, tab_v, sem)

    @pl.when(jnp.logical_not(is_last))
    def _():
        sl = pl.ds(base, CHUNK)
        fsl = pl.ds(base * 3, FCHUNK)
        c = [
            pltpu.async_copy(ne_hbm.at[sl], ne_v, sem),
            pltpu.async_copy(sp_hbm.at[sl], sp_v, sem),
            pltpu.async_copy(lg_hbm.at[sl], lg_v, sem),
            pltpu.async_copy(bt_hbm.at[sl], bt_v, sem),
            pltpu.async_copy(f_hbm.at[fsl], f_v, fsem),
        ]
        for cc in c[:4]:
            cc.wait()

    @pl.when(is_last)
    def _():
        sl = pl.ds(base, TAIL)
        tsl = pl.ds(0, TAIL)
        fsl = pl.ds(base * 3, FTAIL)
        c = [
            pltpu.async_copy(ne_hbm.at[sl], ne_v.at[tsl], sem),
            pltpu.async_copy(sp_hbm.at[sl], sp_v.at[tsl], sem),
            pltpu.async_copy(lg_hbm.at[sl], lg_v.at[tsl], sem),
            pltpu.async_copy(bt_hbm.at[sl], bt_v.at[tsl], sem),
            pltpu.async_copy(f_hbm.at[fsl], f_v.at[pl.ds(0, FTAIL)], fsem),
        ]
        for cc in c[:4]:
            cc.wait()

    zeros16 = jnp.zeros((L,), jnp.float32)

    @plsc.parallel_loop(0, (L * NUM_GRAPHS) // L, unroll=4)
    def _(i):
        acc_v[pl.ds(i * L, L)] = zeros16

    tab_cp.wait()

    lane_base = lax.iota(jnp.int32, L) * NUM_GRAPHS
    nvec = jnp.where(is_last, TAIL_VECS, VECS)

    @plsc.parallel_loop(0, nvec, unroll=8)
    def _(i):
        sl = pl.ds(i * L, L)
        idx = sp_v[sl]
        tv = plsc.load_gather(tab_v, [idx])
        lgv = lg_v[sl]
        e = ne_v[sl] + tv * lgv
        oe_v[sl] = e
        plsc.addupdate_scatter(acc_v, [lane_base + bt_v[sl]], e * lgv)

    @plsc.parallel_loop(0, NUM_GRAPHS // L, unroll=2)
    def _(j):
        s = jnp.zeros((L,), jnp.float32)
        for l in range(L):
            s = s + acc_v[pl.ds(l * NUM_GRAPHS + j * L, L)]
        par_v[pl.ds(j * L, L)] = s

    pltpu.sync_copy(par_v, par_hbm.at[wid])

    @pl.when(jnp.logical_not(is_last))
    def _():
        pltpu.make_async_copy(f_hbm.at[pl.ds(0, FCHUNK)], f_v, fsem).wait()
        ocp = pltpu.async_copy(oe_v, oe_hbm.at[pl.ds(base, CHUNK)], sem)
        fcp = pltpu.async_copy(f_v, fo_hbm.at[pl.ds(base * 3, FCHUNK)], fsem)
        ocp.wait()
        fcp.wait()

    @pl.when(is_last)
    def _():
        pltpu.make_async_copy(
            f_hbm.at[pl.ds(0, FTAIL)], f_v.at[pl.ds(0, FTAIL)], fsem).wait()
        ocp = pltpu.async_copy(
            oe_v.at[pl.ds(0, TAIL)], oe_hbm.at[pl.ds(base, TAIL)], sem)
        fcp = pltpu.async_copy(
            f_v.at[pl.ds(0, FTAIL)], fo_hbm.at[pl.ds(base * 3, FTAIL)], fsem)
        ocp.wait()
        fcp.wait()


@jax.jit
def _sc_call(ne, sp, lg_f, bt, tab_p, f_flat):
    mesh = plsc.VectorSubcoreMesh(core_axis_name="c", subcore_axis_name="s")
    k = functools.partial(
        pl.kernel,
        mesh=mesh,
        compiler_params=pltpu.CompilerParams(needs_layout_passes=False),
        out_type=(
            jax.ShapeDtypeStruct((N_NODES,), jnp.float32),
            jax.ShapeDtypeStruct((NW, NUM_GRAPHS), jnp.float32),
            jax.ShapeDtypeStruct((N_NODES * 3,), jnp.float32),
        ),
        scratch_types=[
            pltpu.VMEM((CHUNK,), jnp.float32),
            pltpu.VMEM((CHUNK,), jnp.int32),
            pltpu.VMEM((CHUNK,), jnp.float32),
            pltpu.VMEM((CHUNK,), jnp.int32),
            pltpu.VMEM((CHUNK,), jnp.float32),
            pltpu.VMEM((TABLE_PAD,), jnp.float32),
            pltpu.VMEM((L * NUM_GRAPHS,), jnp.float32),
            pltpu.VMEM((NUM_GRAPHS,), jnp.float32),
            pltpu.VMEM((FCHUNK,), jnp.float32),
            pltpu.SemaphoreType.DMA,
            pltpu.SemaphoreType.DMA,
        ],
    )(_sc_body)
    return k(ne, sp, lg_f, bt, tab_p, f_flat)


def kernel(node_energy, forces, species, local_or_ghost, batch, ptr, enr_table):
    sp = species.astype(jnp.int32)
    lg_f = local_or_ghost.astype(jnp.float32)
    bt = batch.astype(jnp.int32)
    tab_p = jnp.pad(enr_table, (0, TABLE_PAD - enr_table.shape[0]))
    f_flat = forces.reshape(-1)

    oe, partials, fo = _sc_call(node_energy, sp, lg_f, bt, tab_p, f_flat)

    total_energy_local = partials.sum(axis=0)
    virials = jnp.zeros((1, 3, 3), dtype=node_energy.dtype)
    return (total_energy_local, oe, fo.reshape(N_NODES, 3), virials)


# speedup vs baseline: 5.9182x; 5.9182x over previous
import functools

import jax
import jax.numpy as jnp
from jax import lax
from jax.experimental import pallas as pl
from jax.experimental.pallas import tpu as pltpu
from jax.experimental.pallas import tpu_sc as plsc

N_NODES = 100000
NUM_GRAPHS = 64
NW = 32
CHUNK = 3136
LAST = NW - 1
TAIL = N_NODES - LAST * CHUNK
TABLE_PAD = 128
L = 16
VECS = CHUNK // L
TAIL_VECS = TAIL // L
FCHUNK = CHUNK * 3
FTAIL = TAIL * 3


def _sc_body(ne_hbm, sp_hbm, lg_hbm, bt_hbm, tab_hbm,
             oe_hbm, par_hbm,
             ne_v, sp_v, lg_v, bt_v, oe_v, tab_v, acc_v, par_v,
             sem):
    wid = lax.axis_index("s") * 2 + lax.axis_index("c")
    base = wid * CHUNK
    is_last = wid == LAST

    tab_cp = pltpu.async_copy(tab_hbm, tab_v, sem)

    @pl.when(jnp.logical_not(is_last))
    def _():
        sl = pl.ds(base, CHUNK)
        c = [
            pltpu.async_copy(ne_hbm.at[sl], ne_v, sem),
            pltpu.async_copy(sp_hbm.at[sl], sp_v, sem),
            pltpu.async_copy(lg_hbm.at[sl], lg_v, sem),
            pltpu.async_copy(bt_hbm.at[sl], bt_v, sem),
        ]
        for cc in c:
            cc.wait()

    @pl.when(is_last)
    def _():
        sl = pl.ds(base, TAIL)
        tsl = pl.ds(0, TAIL)
        c = [
            pltpu.async_copy(ne_hbm.at[sl], ne_v.at[tsl], sem),
            pltpu.async_copy(sp_hbm.at[sl], sp_v.at[tsl], sem),
            pltpu.async_copy(lg_hbm.at[sl], lg_v.at[tsl], sem),
            pltpu.async_copy(bt_hbm.at[sl], bt_v.at[tsl], sem),
        ]
        for cc in c:
            cc.wait()

    zeros16 = jnp.zeros((L,), jnp.float32)

    @plsc.parallel_loop(0, (L * NUM_GRAPHS) // L, unroll=4)
    def _(i):
        acc_v[pl.ds(i * L, L)] = zeros16

    tab_cp.wait()

    lane_base = lax.iota(jnp.int32, L) * NUM_GRAPHS
    nvec = jnp.where(is_last, TAIL_VECS, VECS)

    @plsc.parallel_loop(0, nvec, unroll=8)
    def _(i):
        sl = pl.ds(i * L, L)
        idx = sp_v[sl]
        tv = plsc.load_gather(tab_v, [idx])
        lgv = lg_v[sl]
        e = ne_v[sl] + tv * lgv
        oe_v[sl] = e
        plsc.addupdate_scatter(acc_v, [lane_base + bt_v[sl]], e * lgv)

    @plsc.parallel_loop(0, NUM_GRAPHS // L, unroll=2)
    def _(j):
        s = jnp.zeros((L,), jnp.float32)
        for l in range(L):
            s = s + acc_v[pl.ds(l * NUM_GRAPHS + j * L, L)]
        par_v[pl.ds(j * L, L)] = s

    pltpu.sync_copy(par_v, par_hbm.at[wid])

    @pl.when(jnp.logical_not(is_last))
    def _():
        pltpu.sync_copy(oe_v, oe_hbm.at[pl.ds(base, CHUNK)])

    @pl.when(is_last)
    def _():
        pltpu.sync_copy(
            oe_v.at[pl.ds(0, TAIL)], oe_hbm.at[pl.ds(base, TAIL)])


@jax.jit
def _sc_call(ne, sp, lg_f, bt, tab_p):
    mesh = plsc.VectorSubcoreMesh(core_axis_name="c", subcore_axis_name="s")
    k = functools.partial(
        pl.kernel,
        mesh=mesh,
        compiler_params=pltpu.CompilerParams(needs_layout_passes=False),
        out_type=(
            jax.ShapeDtypeStruct((N_NODES,), jnp.float32),
            jax.ShapeDtypeStruct((NW, NUM_GRAPHS), jnp.float32),
        ),
        scratch_types=[
            pltpu.VMEM((CHUNK,), jnp.float32),
            pltpu.VMEM((CHUNK,), jnp.int32),
            pltpu.VMEM((CHUNK,), jnp.float32),
            pltpu.VMEM((CHUNK,), jnp.int32),
            pltpu.VMEM((CHUNK,), jnp.float32),
            pltpu.VMEM((TABLE_PAD,), jnp.float32),
            pltpu.VMEM((L * NUM_GRAPHS,), jnp.float32),
            pltpu.VMEM((NUM_GRAPHS,), jnp.float32),
            pltpu.SemaphoreType.DMA,
        ],
    )(_sc_body)
    return k(ne, sp, lg_f, bt, tab_p)


def kernel(node_energy, forces, species, local_or_ghost, batch, ptr, enr_table):
    sp = species.astype(jnp.int32)
    lg_f = local_or_ghost.astype(jnp.float32)
    bt = batch.astype(jnp.int32)
    tab_p = jnp.pad(enr_table, (0, TABLE_PAD - enr_table.shape[0]))

    oe, partials = _sc_call(node_energy, sp, lg_f, bt, tab_p)

    total_energy_local = partials.sum(axis=0)
    virials = jnp.zeros((1, 3, 3), dtype=node_energy.dtype)
    return (total_energy_local, oe, forces, virials)
